# Initial kernel scaffold; baseline (speedup 1.0000x reference)
import jax
import jax.numpy as jnp
from jax.experimental import pallas as pl


def kernel(inputs, state, rows1, cols1, vals1, rows2, cols2, vals2, W_gate, b_gate, W_cand, b_cand):
    raise NotImplementedError("write your pallas kernel here")


# speedup vs baseline: 2.6198x; 2.6198x over previous
import functools

import jax
import jax.numpy as jnp
from jax import lax
from jax.experimental import pallas as pl
from jax.experimental.pallas import tpu as pltpu
from jax.experimental.pallas import tpu_sc as plsc

N = 10000
DEG = 16
E = N * DEG
B = 16
IN_DIM = 2
U = 64
IN_SIZE = IN_DIM + U
F = IN_SIZE * B

C = 48
NCHUNK = F // C
CHUNK_PER_SC = NCHUNK // 2
NTILE = 16
EDGE_PER_TILE = E // NTILE
GB = 80
NB = EDGE_PER_TILE // GB
ROWS_PER_TILE = N // NTILE

BLK = 3200


def _spmm_sc(rows3, cols3, vals3, x3):
    mesh = plsc.VectorSubcoreMesh(core_axis_name="c", subcore_axis_name="s")

    @functools.partial(
        pl.kernel,
        mesh=mesh,
        compiler_params=pltpu.CompilerParams(use_tc_tiling_on_sc=False),
        out_type=jax.ShapeDtypeStruct((N, NCHUNK, C), jnp.float32),
        scratch_types=[
            pltpu.VMEM_SHARED((N, C), jnp.float32),
            pltpu.VMEM_SHARED((N, C), jnp.float32),
            pltpu.VMEM((NB, GB), jnp.int32),
            pltpu.VMEM((NB, GB), jnp.int32),
            pltpu.VMEM((NB, GB), jnp.float32),
            pltpu.VMEM((GB, C), jnp.float32),
            pltpu.VMEM((ROWS_PER_TILE, C), jnp.float32),
        ],
    )
    def k(rows_hbm, cols_hbm, vals_hbm, x_hbm, y_hbm,
          xc_sp, out_sp, rows_t, cols_t, vals_t, gbuf, zbuf):
        cid = lax.axis_index("c")
        sid = lax.axis_index("s")
        rslice = pl.ds(sid * ROWS_PER_TILE, ROWS_PER_TILE)

        pltpu.sync_copy(rows_hbm.at[sid], rows_t)
        pltpu.sync_copy(cols_hbm.at[sid], cols_t)
        pltpu.sync_copy(vals_hbm.at[sid], vals_t)

        zvec = jnp.zeros((16,), jnp.float32)

        def zrow(i, carry):
            for kk in range(C // 16):
                zbuf[i, pl.ds(kk * 16, 16)] = zvec
            return carry

        lax.fori_loop(0, ROWS_PER_TILE, zrow, 0)

        def chunk_body(j, carry):
            ci = cid * CHUNK_PER_SC + j
            pltpu.sync_copy(x_hbm.at[rslice, ci], xc_sp.at[rslice])
            pltpu.sync_copy(zbuf, out_sp.at[rslice])
            plsc.subcore_barrier()

            def batch_body(b, bcarry):
                pltpu.sync_copy(xc_sp.at[cols_t.at[b]], gbuf)

                def scale(g2, scarry):
                    vblock = vals_t[b, pl.ds(g2 * 16, 16)]
                    for l in range(16):
                        val = vblock[l]
                        row = g2 * 16 + l
                        for kk in range(C // 16):
                            sl = pl.ds(kk * 16, 16)
                            gbuf[row, sl] = gbuf[row, sl] * val
                    return scarry

                lax.fori_loop(0, GB // 16, scale, 0)
                pltpu.sync_copy(gbuf, out_sp.at[rows_t.at[b]], add=True)
                return bcarry

            lax.fori_loop(0, NB, batch_body, 0)
            plsc.subcore_barrier()
            pltpu.sync_copy(out_sp.at[rslice], y_hbm.at[rslice, ci])
            return carry

        lax.fori_loop(0, CHUNK_PER_SC, chunk_body, 0)

    return k(rows3, cols3, vals3, x3)


def _dot(a, b):
    return jax.lax.dot_general(a, b, (((1,), (0,)), ((), ())),
                               preferred_element_type=jnp.float32)


def _gate_body(x0, y1a, y2a, y1b, y2b, w0, w1, w2, w3, w4, bg, st,
               xc_out, u_out):
    acc = _dot(x0[...], w0[...]) + bg[...]
    acc += _dot(y1a[...], w1[...])
    acc += _dot(y2a[...], w2[...])
    acc += _dot(y1b[...], w3[...])
    acc += _dot(y2b[...], w4[...])
    g = jax.nn.sigmoid(acc)
    r = g[:, :U]
    u = g[:, U:]
    xc_out[:, 0:IN_DIM] = x0[:, 0:IN_DIM]
    xc_out[:, IN_DIM:IN_SIZE] = r * st[...]
    u_out[...] = u


def _cand_body(x0c, y1a, y2a, y1b, y2b, w0, w1, w2, w3, w4, bc, u, st,
               out_ref):
    acc = _dot(x0c[...], w0[...]) + bc[...]
    acc += _dot(y1a[...], w1[...])
    acc += _dot(y2a[...], w2[...])
    acc += _dot(y1b[...], w3[...])
    acc += _dot(y2b[...], w4[...])
    c = jnp.tanh(acc)
    uu = u[...]
    out_ref[...] = uu * st[...] + (1.0 - uu) * c


def _row_spec(cols):
    return pl.BlockSpec((BLK, cols), lambda i: (i, 0))


def _full_spec(r, cols):
    return pl.BlockSpec((r, cols), lambda i: (0, 0))


def _gate_tc(x0, y1a, y2a, y1b, y2b, wg, bg, st):
    grid = (x0.shape[0] // BLK,)
    in_specs = ([_row_spec(IN_SIZE)] * 5
                + [_full_spec(IN_SIZE, 2 * U)] * 5
                + [_full_spec(1, 2 * U), _row_spec(U)])
    return pl.pallas_call(
        _gate_body,
        grid=grid,
        in_specs=in_specs,
        out_specs=[_row_spec(IN_SIZE), _row_spec(U)],
        out_shape=[jax.ShapeDtypeStruct((x0.shape[0], IN_SIZE), jnp.float32),
                   jax.ShapeDtypeStruct((x0.shape[0], U), jnp.float32)],
    )(x0, y1a, y2a, y1b, y2b, wg[0], wg[1], wg[2], wg[3], wg[4], bg, st)


def _cand_tc(x0c, y1a, y2a, y1b, y2b, wc, bc, u, st):
    grid = (x0c.shape[0] // BLK,)
    in_specs = ([_row_spec(IN_SIZE)] * 5
                + [_full_spec(IN_SIZE, U)] * 5
                + [_full_spec(1, U), _row_spec(U), _row_spec(U)])
    return pl.pallas_call(
        _cand_body,
        grid=grid,
        in_specs=in_specs,
        out_specs=_row_spec(U),
        out_shape=jax.ShapeDtypeStruct((x0c.shape[0], U), jnp.float32),
    )(x0c, y1a, y2a, y1b, y2b, wc[0], wc[1], wc[2], wc[3], wc[4], bc, u, st)


def _fold_weights(W):
    Wr = W.reshape(IN_SIZE, 5, W.shape[1])
    w0 = Wr[:, 0] - Wr[:, 2] - Wr[:, 4]
    return (w0, Wr[:, 1], 2.0 * Wr[:, 2], Wr[:, 3], 2.0 * Wr[:, 4])


def kernel(inputs, state, rows1, cols1, vals1, rows2, cols2, vals2,
           W_gate, b_gate, W_cand, b_cand):
    in_t = jnp.transpose(inputs.reshape(B, N, IN_DIM), (1, 0, 2))
    st_t = jnp.transpose(state.reshape(B, N, U), (1, 0, 2))
    x0 = jnp.concatenate([in_t, st_t], axis=2)
    x0_sc = x0.reshape(N, NCHUNK, C)
    x0_tc = x0.reshape(N * B, IN_SIZE)
    st_tc = st_t.reshape(N * B, U)

    e1 = (rows1.reshape(NTILE, NB, GB), cols1.reshape(NTILE, NB, GB),
          vals1.reshape(NTILE, NB, GB))
    e2 = (rows2.reshape(NTILE, NB, GB), cols2.reshape(NTILE, NB, GB),
          vals2.reshape(NTILE, NB, GB))

    wg = _fold_weights(W_gate)
    wc = _fold_weights(W_cand)
    bg = b_gate.reshape(1, 2 * U)
    bc = b_cand.reshape(1, U)

    def diffuse(x_sc):
        y1a = _spmm_sc(*e1, x_sc)
        y2a = _spmm_sc(*e1, y1a)
        y1b = _spmm_sc(*e2, x_sc)
        y2b = _spmm_sc(*e2, y1b)
        flat = lambda t: t.reshape(N * B, IN_SIZE)
        return flat(y1a), flat(y2a), flat(y1b), flat(y2b)

    g1a, g2a, g1b, g2b = diffuse(x0_sc)
    xc_cand, u = _gate_tc(x0_tc, g1a, g2a, g1b, g2b, wg, bg, st_tc)

    c1a, c2a, c1b, c2b = diffuse(xc_cand.reshape(N, NCHUNK, C))
    new_t = _cand_tc(xc_cand, c1a, c2a, c1b, c2b, wc, bc, u, st_tc)

    new_state = jnp.transpose(new_t.reshape(N, B, U), (1, 0, 2))
    return new_state.reshape(B, N * U)


# speedup vs baseline: 3.5384x; 1.3506x over previous
import functools

import jax
import jax.numpy as jnp
from jax import lax
from jax.experimental import pallas as pl
from jax.experimental.pallas import tpu as pltpu
from jax.experimental.pallas import tpu_sc as plsc

N = 10000
DEG = 16
E = N * DEG
B = 16
IN_DIM = 2
U = 64
IN_SIZE = IN_DIM + U
F = IN_SIZE * B

C = 48
NCHUNK = F // C
CHUNK_PER_SC = NCHUNK // 2
NTILE = 16
EDGE_PER_TILE = E // NTILE
GB = 400
NB = EDGE_PER_TILE // GB
ROWS_PER_TILE = N // NTILE
ZROWS = 25

BLK = 3200


def _spmm_sc(rows3, cols3, vals3, x3):
    mesh = plsc.VectorSubcoreMesh(core_axis_name="c", subcore_axis_name="s")

    @functools.partial(
        pl.kernel,
        mesh=mesh,
        compiler_params=pltpu.CompilerParams(use_tc_tiling_on_sc=False),
        out_type=jax.ShapeDtypeStruct((N, NCHUNK, C), jnp.float32),
        scratch_types=[
            pltpu.VMEM_SHARED((N, C), jnp.float32),
            pltpu.VMEM_SHARED((N, C), jnp.float32),
            pltpu.VMEM((NB, GB), jnp.int32),
            pltpu.VMEM((NB, GB), jnp.int32),
            pltpu.VMEM((NB, GB), jnp.float32),
            pltpu.VMEM((GB, C), jnp.float32),
            pltpu.VMEM((GB, C), jnp.float32),
            pltpu.VMEM((ZROWS, C), jnp.float32),
            pltpu.SemaphoreType.DMA,
            pltpu.SemaphoreType.DMA,
        ],
    )
    def k(rows_hbm, cols_hbm, vals_hbm, x_hbm, y_hbm,
          xc_sp, out_sp, rows_t, cols_t, vals_t, gbuf0, gbuf1, zbuf,
          ssem0, ssem1):
        cid = lax.axis_index("c")
        sid = lax.axis_index("s")
        rslice = pl.ds(sid * ROWS_PER_TILE, ROWS_PER_TILE)

        pltpu.sync_copy(rows_hbm.at[sid], rows_t)
        pltpu.sync_copy(cols_hbm.at[sid], cols_t)
        pltpu.sync_copy(vals_hbm.at[sid], vals_t)

        zvec = jnp.zeros((16,), jnp.float32)

        def zrow(i, carry):
            for kk in range(C // 16):
                zbuf[i, pl.ds(kk * 16, 16)] = zvec
            return carry

        lax.fori_loop(0, ZROWS, zrow, 0)

        def chunk_body(j, carry):
            ci = cid * CHUNK_PER_SC + j
            pltpu.sync_copy(x_hbm.at[rslice, ci], xc_sp.at[rslice])

            def zinit(z, zcarry):
                pltpu.sync_copy(
                    zbuf,
                    out_sp.at[pl.ds(sid * ROWS_PER_TILE + z * ZROWS, ZROWS)])
                return zcarry

            lax.fori_loop(0, ROWS_PER_TILE // ZROWS, zinit, 0)
            plsc.subcore_barrier()

            def scale(gb, b):
                def scale_grp(g2, scarry):
                    vblock = vals_t[b, pl.ds(g2 * 16, 16)]
                    for l in range(16):
                        val = vblock[l]
                        row = g2 * 16 + l
                        for kk in range(C // 16):
                            sl = pl.ds(kk * 16, 16)
                            gb[row, sl] = gb[row, sl] * val
                    return scarry

                lax.fori_loop(0, GB // 16, scale_grp, 0)

            def batch_pair(i, bcarry):
                for which, (gb, sem) in enumerate(
                        ((gbuf0, ssem0), (gbuf1, ssem1))):
                    b = 2 * i + which

                    @pl.when(i > 0)
                    def _wait_prev():
                        pltpu.make_async_copy(
                            gb, out_sp.at[rows_t.at[b - 2]], sem).wait()

                    pltpu.sync_copy(xc_sp.at[cols_t.at[b]], gb)
                    scale(gb, b)
                    pltpu.async_copy(
                        gb, out_sp.at[rows_t.at[b]], sem, add=True)
                return bcarry

            lax.fori_loop(0, NB // 2, batch_pair, 0)
            bt = NB - 1
            pltpu.make_async_copy(
                gbuf0, out_sp.at[rows_t.at[bt - 2]], ssem0).wait()
            pltpu.sync_copy(xc_sp.at[cols_t.at[bt]], gbuf0)
            scale(gbuf0, bt)
            pltpu.async_copy(gbuf0, out_sp.at[rows_t.at[bt]], ssem0, add=True)
            pltpu.make_async_copy(
                gbuf1, out_sp.at[rows_t.at[bt - 1]], ssem1).wait()
            pltpu.make_async_copy(
                gbuf0, out_sp.at[rows_t.at[bt]], ssem0).wait()
            plsc.subcore_barrier()
            pltpu.sync_copy(out_sp.at[rslice], y_hbm.at[rslice, ci])
            return carry

        lax.fori_loop(0, CHUNK_PER_SC, chunk_body, 0)

    return k(rows3, cols3, vals3, x3)


def _dot(a, b):
    return jax.lax.dot_general(a, b, (((1,), (0,)), ((), ())),
                               preferred_element_type=jnp.float32)


def _gate_body(x0, y1a, y2a, y1b, y2b, w0, w1, w2, w3, w4, bg, st,
               xc_out, u_out):
    acc = _dot(x0[...], w0[...]) + bg[...]
    acc += _dot(y1a[...], w1[...])
    acc += _dot(y2a[...], w2[...])
    acc += _dot(y1b[...], w3[...])
    acc += _dot(y2b[...], w4[...])
    g = jax.nn.sigmoid(acc)
    r = g[:, :U]
    u = g[:, U:]
    xc_out[:, 0:IN_DIM] = x0[:, 0:IN_DIM]
    xc_out[:, IN_DIM:IN_SIZE] = r * st[...]
    u_out[...] = u


def _cand_body(x0c, y1a, y2a, y1b, y2b, w0, w1, w2, w3, w4, bc, u, st,
               out_ref):
    acc = _dot(x0c[...], w0[...]) + bc[...]
    acc += _dot(y1a[...], w1[...])
    acc += _dot(y2a[...], w2[...])
    acc += _dot(y1b[...], w3[...])
    acc += _dot(y2b[...], w4[...])
    c = jnp.tanh(acc)
    uu = u[...]
    out_ref[...] = uu * st[...] + (1.0 - uu) * c


def _row_spec(cols):
    return pl.BlockSpec((BLK, cols), lambda i: (i, 0))


def _full_spec(r, cols):
    return pl.BlockSpec((r, cols), lambda i: (0, 0))


def _gate_tc(x0, y1a, y2a, y1b, y2b, wg, bg, st):
    grid = (x0.shape[0] // BLK,)
    in_specs = ([_row_spec(IN_SIZE)] * 5
                + [_full_spec(IN_SIZE, 2 * U)] * 5
                + [_full_spec(1, 2 * U), _row_spec(U)])
    return pl.pallas_call(
        _gate_body,
        grid=grid,
        in_specs=in_specs,
        out_specs=[_row_spec(IN_SIZE), _row_spec(U)],
        out_shape=[jax.ShapeDtypeStruct((x0.shape[0], IN_SIZE), jnp.float32),
                   jax.ShapeDtypeStruct((x0.shape[0], U), jnp.float32)],
    )(x0, y1a, y2a, y1b, y2b, wg[0], wg[1], wg[2], wg[3], wg[4], bg, st)


def _cand_tc(x0c, y1a, y2a, y1b, y2b, wc, bc, u, st):
    grid = (x0c.shape[0] // BLK,)
    in_specs = ([_row_spec(IN_SIZE)] * 5
                + [_full_spec(IN_SIZE, U)] * 5
                + [_full_spec(1, U), _row_spec(U), _row_spec(U)])
    return pl.pallas_call(
        _cand_body,
        grid=grid,
        in_specs=in_specs,
        out_specs=_row_spec(U),
        out_shape=jax.ShapeDtypeStruct((x0c.shape[0], U), jnp.float32),
    )(x0c, y1a, y2a, y1b, y2b, wc[0], wc[1], wc[2], wc[3], wc[4], bc, u, st)


def _fold_weights(W):
    Wr = W.reshape(IN_SIZE, 5, W.shape[1])
    w0 = Wr[:, 0] - Wr[:, 2] - Wr[:, 4]
    return (w0, Wr[:, 1], 2.0 * Wr[:, 2], Wr[:, 3], 2.0 * Wr[:, 4])


def kernel(inputs, state, rows1, cols1, vals1, rows2, cols2, vals2,
           W_gate, b_gate, W_cand, b_cand):
    in_t = jnp.transpose(inputs.reshape(B, N, IN_DIM), (1, 0, 2))
    st_t = jnp.transpose(state.reshape(B, N, U), (1, 0, 2))
    x0 = jnp.concatenate([in_t, st_t], axis=2)
    x0_sc = x0.reshape(N, NCHUNK, C)
    x0_tc = x0.reshape(N * B, IN_SIZE)
    st_tc = st_t.reshape(N * B, U)

    e1 = (rows1.reshape(NTILE, NB, GB), cols1.reshape(NTILE, NB, GB),
          vals1.reshape(NTILE, NB, GB))
    e2 = (rows2.reshape(NTILE, NB, GB), cols2.reshape(NTILE, NB, GB),
          vals2.reshape(NTILE, NB, GB))

    wg = _fold_weights(W_gate)
    wc = _fold_weights(W_cand)
    bg = b_gate.reshape(1, 2 * U)
    bc = b_cand.reshape(1, U)

    def diffuse(x_sc):
        y1a = _spmm_sc(*e1, x_sc)
        y2a = _spmm_sc(*e1, y1a)
        y1b = _spmm_sc(*e2, x_sc)
        y2b = _spmm_sc(*e2, y1b)
        flat = lambda t: t.reshape(N * B, IN_SIZE)
        return flat(y1a), flat(y2a), flat(y1b), flat(y2b)

    g1a, g2a, g1b, g2b = diffuse(x0_sc)
    xc_cand, u = _gate_tc(x0_tc, g1a, g2a, g1b, g2b, wg, bg, st_tc)

    c1a, c2a, c1b, c2b = diffuse(xc_cand.reshape(N, NCHUNK, C))
    new_t = _cand_tc(xc_cand, c1a, c2a, c1b, c2b, wc, bc, u, st_tc)

    new_state = jnp.transpose(new_t.reshape(N, B, U), (1, 0, 2))
    return new_state.reshape(B, N * U)
